# Initial kernel scaffold; baseline (speedup 1.0000x reference)
import jax
import jax.numpy as jnp
from jax.experimental import pallas as pl


def kernel(nodes, edges, senders, receivers, node_embed, edge_embed, step_params, decoder, train):
    raise NotImplementedError("write your pallas kernel here")


# speedup vs baseline: 3.1592x; 3.1592x over previous
import jax
import jax.numpy as jnp
from jax import lax
from jax.experimental import pallas as pl
from jax.experimental.pallas import tpu as pltpu
from jax.experimental.pallas import tpu_sc as plsc

NN = 10000
NE = 160000
D = 128
EPS = 1e-6

NC, NS = 2, 16
CHUNK = 80
CPT = NE // (NS * CHUNK)
NPT8 = (NN // NS) // 8 * 8
NREM = NN - NS * NPT8

_SC_MESH = plsc.VectorSubcoreMesh(core_axis_name="c", subcore_axis_name="s",
                                  num_cores=NC, num_subcores=NS)


def _sc_gather_body(xs_hbm, xr_hbm, s_hbm, r_hbm, gs_hbm, gr_hbm,
                    idx_v, rows_v, sem):
    cid = lax.axis_index("c")
    sid = lax.axis_index("s")

    def run(tab, idx3d, out):
        pltpu.sync_copy(idx3d.at[sid], idx_v)

        def body(j, carry):
            pltpu.async_copy(tab.at[idx_v.at[j]], rows_v, sem).wait()
            pltpu.sync_copy(rows_v, out.at[pl.ds((sid * CPT + j) * CHUNK, CHUNK)])
            return carry

        lax.fori_loop(0, CPT, body, 0)

    @pl.when(cid == 0)
    def _():
        run(xs_hbm, s_hbm, gs_hbm)

    @pl.when(cid == 1)
    def _():
        run(xr_hbm, r_hbm, gr_hbm)


_sc_gather = pl.kernel(
    _sc_gather_body,
    out_type=[jax.ShapeDtypeStruct((NE, D), jnp.float32),
              jax.ShapeDtypeStruct((NE, D), jnp.float32)],
    mesh=_SC_MESH,
    scratch_types=[pltpu.VMEM((CPT, CHUNK), jnp.int32),
                   pltpu.VMEM((CHUNK, D), jnp.float32),
                   pltpu.SemaphoreType.DMA],
)


def _sc_scatter_body(e_hbm, s_hbm, r_hbm, zero_hbm, aggs_hbm, aggr_hbm,
                     idx_v, rows_v, acc):
    cid = lax.axis_index("c")
    sid = lax.axis_index("s")
    pltpu.sync_copy(zero_hbm.at[pl.ds(sid * NPT8, NPT8)],
                    acc.at[pl.ds(sid * NPT8, NPT8)])

    @pl.when(sid == NS - 1)
    def _():
        pltpu.sync_copy(zero_hbm.at[pl.ds(NS * NPT8, NREM)],
                        acc.at[pl.ds(NS * NPT8, NREM)])

    plsc.subcore_barrier()

    def run(idx3d, out):
        pltpu.sync_copy(idx3d.at[sid], idx_v)

        def body(j, carry):
            pltpu.sync_copy(e_hbm.at[pl.ds((sid * CPT + j) * CHUNK, CHUNK)], rows_v)
            pltpu.sync_copy(rows_v, acc.at[idx_v.at[j]], add=True)
            return carry

        lax.fori_loop(0, CPT, body, 0)
        plsc.subcore_barrier()
        pltpu.sync_copy(acc.at[pl.ds(sid * NPT8, NPT8)],
                        out.at[pl.ds(sid * NPT8, NPT8)])

        @pl.when(sid == NS - 1)
        def _():
            pltpu.sync_copy(acc.at[pl.ds(NS * NPT8, NREM)],
                            out.at[pl.ds(NS * NPT8, NREM)])

    @pl.when(cid == 0)
    def _():
        run(s_hbm, aggs_hbm)

    @pl.when(cid == 1)
    def _():
        run(r_hbm, aggr_hbm)


_sc_scatter = pl.kernel(
    _sc_scatter_body,
    out_type=[jax.ShapeDtypeStruct((NN, D), jnp.float32),
              jax.ShapeDtypeStruct((NN, D), jnp.float32)],
    mesh=_SC_MESH,
    scratch_types=[pltpu.VMEM((CPT, CHUNK), jnp.int32),
                   pltpu.VMEM((CHUNK, D), jnp.float32),
                   pltpu.VMEM_SHARED((NN, D), jnp.float32)],
)


def _ln_relu(h, scale, bias):
    mean = jnp.mean(h, axis=-1, keepdims=True)
    var = jnp.mean(jnp.square(h - mean), axis=-1, keepdims=True)
    return jnp.maximum((h - mean) * lax.rsqrt(var + EPS) * scale + bias, 0.0)


def _dot(a, b):
    return jnp.dot(a, b, preferred_element_type=jnp.float32)


BE = 2000
GE = NE // BE
BN = 2000
GN = NN // BN

_blk = lambda r: pl.BlockSpec((r, D), lambda i: (i, 0))
_full = lambda s: pl.BlockSpec(s, lambda i: (0, 0))


def _embed_nodes_body(n_ref, Wn, bn, Ws, Wr, x_out, xs_out, xr_out):
    x = _dot(n_ref[...], Wn[...]) + bn[...]
    x_out[...] = x
    xs_out[...] = _dot(x, Ws[...])
    xr_out[...] = _dot(x, Wr[...])


def _embed_nodes(nodes, Wn, bn, Ws, Wr):
    return pl.pallas_call(
        _embed_nodes_body,
        grid=(GN,),
        in_specs=[_blk(BN), _full((D, D)), _full((1, D)), _full((D, D)), _full((D, D))],
        out_specs=[_blk(BN), _blk(BN), _blk(BN)],
        out_shape=[jax.ShapeDtypeStruct((NN, D), jnp.float32)] * 3,
    )(nodes, Wn, bn, Ws, Wr)


def _embed_edges_body(e_ref, We, be, out_ref):
    out_ref[...] = _dot(e_ref[...], We[...]) + be[...]


def _embed_edges(edges, We, be):
    de = edges.shape[-1]
    return pl.pallas_call(
        _embed_edges_body,
        grid=(GE,),
        in_specs=[pl.BlockSpec((BE, de), lambda i: (i, 0)),
                  _full((de, D)), _full((1, D))],
        out_specs=_blk(BE),
        out_shape=jax.ShapeDtypeStruct((NE, D), jnp.float32),
    )(edges, We, be)


def _edge_body(e_ref, gs_ref, gr_ref, g_ref, We, Wg, b, scale, bias,
               out_ref, agg_ref):
    c = _dot(g_ref[...], Wg[...]) + b[...]
    h = _dot(e_ref[...], We[...]) + gs_ref[...] + gr_ref[...] + c
    en = _ln_relu(h, scale[...], bias[...])
    out_ref[...] = en

    @pl.when(pl.program_id(0) == 0)
    def _():
        agg_ref[...] = jnp.zeros_like(agg_ref)

    agg_ref[...] += jnp.sum(en, axis=0, keepdims=True)


def _edge_update(e, gs, gr, g, We, Wg, b, scale, bias):
    return pl.pallas_call(
        _edge_body,
        grid=(GE,),
        in_specs=[_blk(BE), _blk(BE), _blk(BE), _full((1, D)),
                  _full((D, D)), _full((D, D)),
                  _full((1, D)), _full((1, D)), _full((1, D))],
        out_specs=[_blk(BE), _full((1, D))],
        out_shape=[jax.ShapeDtypeStruct((NE, D), jnp.float32),
                   jax.ShapeDtypeStruct((1, D), jnp.float32)],
        compiler_params=pltpu.CompilerParams(dimension_semantics=("arbitrary",)),
    )(e, gs, gr, g, We, Wg, b, scale, bias)


def _node_body(x_ref, as_ref, ar_ref, g_ref, Vx, Vas, Var, Vg, b, scale, bias,
               Wsn, Wrn, x_out, agg_ref, xs_out, xr_out):
    c = _dot(g_ref[...], Vg[...]) + b[...]
    h = (_dot(x_ref[...], Vx[...]) + _dot(as_ref[...], Vas[...])
         + _dot(ar_ref[...], Var[...]) + c)
    xn = _ln_relu(h, scale[...], bias[...])
    x_out[...] = xn

    @pl.when(pl.program_id(0) == 0)
    def _():
        agg_ref[...] = jnp.zeros_like(agg_ref)

    agg_ref[...] += jnp.sum(xn, axis=0, keepdims=True)
    xs_out[...] = _dot(xn, Wsn[...])
    xr_out[...] = _dot(xn, Wrn[...])


def _node_update(x, aggs, aggr, g, Vx, Vas, Var, Vg, b, scale, bias, Wsn, Wrn):
    return pl.pallas_call(
        _node_body,
        grid=(GN,),
        in_specs=[_blk(BN), _blk(BN), _blk(BN), _full((1, D)),
                  _full((D, D)), _full((D, D)), _full((D, D)), _full((D, D)),
                  _full((1, D)), _full((1, D)), _full((1, D)),
                  _full((D, D)), _full((D, D))],
        out_specs=[_blk(BN), _full((1, D)), _blk(BN), _blk(BN)],
        out_shape=[jax.ShapeDtypeStruct((NN, D), jnp.float32),
                   jax.ShapeDtypeStruct((1, D), jnp.float32),
                   jax.ShapeDtypeStruct((NN, D), jnp.float32),
                   jax.ShapeDtypeStruct((NN, D), jnp.float32)],
        compiler_params=pltpu.CompilerParams(dimension_semantics=("arbitrary",)),
    )(x, aggs, aggr, g, Vx, Vas, Var, Vg, b, scale, bias, Wsn, Wrn)


def _node_last_body(x_ref, as_ref, ar_ref, g_ref, Vx, Vas, Var, Vg, b, scale,
                    bias, x_out, agg_ref):
    c = _dot(g_ref[...], Vg[...]) + b[...]
    h = (_dot(x_ref[...], Vx[...]) + _dot(as_ref[...], Vas[...])
         + _dot(ar_ref[...], Var[...]) + c)
    xn = _ln_relu(h, scale[...], bias[...])
    x_out[...] = xn

    @pl.when(pl.program_id(0) == 0)
    def _():
        agg_ref[...] = jnp.zeros_like(agg_ref)

    agg_ref[...] += jnp.sum(xn, axis=0, keepdims=True)


def _node_update_last(x, aggs, aggr, g, Vx, Vas, Var, Vg, b, scale, bias):
    return pl.pallas_call(
        _node_last_body,
        grid=(GN,),
        in_specs=[_blk(BN), _blk(BN), _blk(BN), _full((1, D)),
                  _full((D, D)), _full((D, D)), _full((D, D)), _full((D, D)),
                  _full((1, D)), _full((1, D)), _full((1, D))],
        out_specs=[_blk(BN), _full((1, D))],
        out_shape=[jax.ShapeDtypeStruct((NN, D), jnp.float32),
                   jax.ShapeDtypeStruct((1, D), jnp.float32)],
        compiler_params=pltpu.CompilerParams(dimension_semantics=("arbitrary",)),
    )(x, aggs, aggr, g, Vx, Vas, Var, Vg, b, scale, bias)


def _global_body(na_ref, ea_ref, g_ref, W_ref, b, scale, bias, out_ref):
    W = W_ref[...]
    h = (_dot(na_ref[...], W[0:D]) + _dot(ea_ref[...], W[D:2 * D])
         + _dot(g_ref[...], W[2 * D:3 * D]) + b[...])
    out_ref[...] = _ln_relu(h, scale[...], bias[...])


def _global_update(na, ea, g, W, b, scale, bias):
    return pl.pallas_call(
        _global_body,
        out_shape=jax.ShapeDtypeStruct((1, D), jnp.float32),
    )(na, ea, g, W, b, scale, bias)


def _decode_body(g_ref, Wd, bd, out_ref):
    out_ref[...] = _dot(g_ref[...], Wd[...]) + bd[...]


def _decode(g, Wd, bd):
    return pl.pallas_call(
        _decode_body,
        out_shape=jax.ShapeDtypeStruct((1, D), jnp.float32),
    )(g, Wd, bd)


def kernel(nodes, edges, senders, receivers, node_embed, edge_embed,
           step_params, decoder, train=False):
    s2d = senders.reshape(NS, CPT, CHUNK)
    r2d = receivers.reshape(NS, CPT, CHUNK)
    zeros = jnp.zeros((NN, D), jnp.float32)
    g = jnp.zeros((1, D), jnp.float32)

    def row(v):
        return v.reshape(1, D)

    W0 = step_params[0]['edge'][0]
    x, xs, xr = _embed_nodes(nodes, node_embed[0], row(node_embed[1]),
                             W0[D:2 * D], W0[2 * D:3 * D])
    e = _embed_edges(edges, edge_embed[0], row(edge_embed[1]))

    for t in range(len(step_params)):
        ep = step_params[t]['edge']
        npar = step_params[t]['node']
        gp = step_params[t]['global']
        W = ep[0]
        gs, gr = _sc_gather(xs, xr, s2d, r2d)
        e, e_agg = _edge_update(e, gs, gr, g, W[0:D], W[3 * D:4 * D],
                                row(ep[1]), row(ep[2]), row(ep[3]))
        agg_s, agg_r = _sc_scatter(e, s2d, r2d, zeros)
        V = npar[0]
        if t + 1 < len(step_params):
            Wn = step_params[t + 1]['edge'][0]
            x, n_agg, xs, xr = _node_update(
                x, agg_s, agg_r, g, V[0:D], V[D:2 * D], V[2 * D:3 * D],
                V[3 * D:4 * D], row(npar[1]), row(npar[2]), row(npar[3]),
                Wn[D:2 * D], Wn[2 * D:3 * D])
        else:
            x, n_agg = _node_update_last(
                x, agg_s, agg_r, g, V[0:D], V[D:2 * D], V[2 * D:3 * D],
                V[3 * D:4 * D], row(npar[1]), row(npar[2]), row(npar[3]))
        g = _global_update(n_agg, e_agg, g, gp[0], row(gp[1]), row(gp[2]),
                           row(gp[3]))

    return _decode(g, decoder[0], row(decoder[1]))


# speedup vs baseline: 4.0752x; 1.2900x over previous
import jax
import jax.numpy as jnp
from jax import lax
from jax.experimental import pallas as pl
from jax.experimental.pallas import tpu as pltpu
from jax.experimental.pallas import tpu_sc as plsc

NN = 10000
NE = 160000
D = 128
EPS = 1e-6

NC, NS = 2, 16
CHUNK = 80
CPT = NE // (NS * CHUNK)
NPT8 = (NN // NS) // 8 * 8
NREM = NN - NS * NPT8

_SC_MESH = plsc.VectorSubcoreMesh(core_axis_name="c", subcore_axis_name="s",
                                  num_cores=NC, num_subcores=NS)


K = 5
KC = K * CHUNK
G = CPT // K
P = (G - 1) // 2
PS = (CPT - 1) // 2


def _sc_gather_body(xs_hbm, xr_hbm, s_hbm, r_hbm, gs_hbm, gr_hbm,
                    idx_v, rows_a, rows_b, gsa, gsb, wsa, wsb):
    cid = lax.axis_index("c")
    sid = lax.axis_index("s")

    def run(tab, idx3d, out):
        pltpu.sync_copy(idx3d.at[sid], idx_v)

        def fire_g(g, buf, sem):
            for k in range(K):
                pltpu.async_copy(tab.at[idx_v.at[g * K + k]],
                                 buf.at[pl.ds(k * CHUNK, CHUNK)], sem)

        def drain_g(buf, sem):
            pltpu.make_async_copy(tab.at[pl.ds(0, KC)], buf, sem).wait()

        def fire_w(g, buf, sem):
            pltpu.async_copy(buf, out.at[pl.ds((sid * G + g) * KC, KC)], sem)

        def drain_w(buf, sem):
            pltpu.make_async_copy(buf, out.at[pl.ds(0, KC)], sem).wait()

        fire_g(0, rows_a, gsa)

        def body(p, carry):
            drain_g(rows_a, gsa)
            fire_w(2 * p, rows_a, wsa)

            @pl.when(p > 0)
            def _():
                drain_w(rows_b, wsb)

            fire_g(2 * p + 1, rows_b, gsb)
            drain_g(rows_b, gsb)
            fire_w(2 * p + 1, rows_b, wsb)
            drain_w(rows_a, wsa)
            fire_g(2 * p + 2, rows_a, gsa)
            return carry

        lax.fori_loop(0, P, body, 0)
        drain_g(rows_a, gsa)
        fire_w(G - 1, rows_a, wsa)
        drain_w(rows_b, wsb)
        drain_w(rows_a, wsa)

    @pl.when(cid == 0)
    def _():
        run(xs_hbm, s_hbm, gs_hbm)

    @pl.when(cid == 1)
    def _():
        run(xr_hbm, r_hbm, gr_hbm)


_sc_gather = pl.kernel(
    _sc_gather_body,
    out_type=[jax.ShapeDtypeStruct((NE, D), jnp.float32),
              jax.ShapeDtypeStruct((NE, D), jnp.float32)],
    mesh=_SC_MESH,
    scratch_types=[pltpu.VMEM((CPT, CHUNK), jnp.int32),
                   pltpu.VMEM((KC, D), jnp.float32),
                   pltpu.VMEM((KC, D), jnp.float32),
                   pltpu.SemaphoreType.DMA,
                   pltpu.SemaphoreType.DMA,
                   pltpu.SemaphoreType.DMA,
                   pltpu.SemaphoreType.DMA],
)


def _sc_scatter_body(e_hbm, s_hbm, r_hbm, zero_hbm, aggs_hbm, aggr_hbm,
                     idx_v, rows_a, rows_b, gsa, gsb, wsa, wsb, acc):
    cid = lax.axis_index("c")
    sid = lax.axis_index("s")
    pltpu.sync_copy(zero_hbm.at[pl.ds(sid * NPT8, NPT8)],
                    acc.at[pl.ds(sid * NPT8, NPT8)])

    @pl.when(sid == NS - 1)
    def _():
        pltpu.sync_copy(zero_hbm.at[pl.ds(NS * NPT8, NREM)],
                        acc.at[pl.ds(NS * NPT8, NREM)])

    plsc.subcore_barrier()

    def run(idx3d, out):
        pltpu.sync_copy(idx3d.at[sid], idx_v)

        def fire_r(j, buf, sem):
            pltpu.async_copy(e_hbm.at[pl.ds((sid * CPT + j) * CHUNK, CHUNK)],
                             buf, sem)

        def drain_r(buf, sem):
            pltpu.make_async_copy(e_hbm.at[pl.ds(0, CHUNK)], buf, sem).wait()

        def fire_s(j, buf, sem):
            pltpu.async_copy(buf, acc.at[idx_v.at[j]], sem, add=True)

        fire_r(0, rows_a, gsa)

        def body(p, carry):
            drain_r(rows_a, gsa)
            fire_s(2 * p, rows_a, wsa)

            @pl.when(p > 0)
            def _():
                drain_r(rows_b, wsb)

            fire_r(2 * p + 1, rows_b, gsb)
            drain_r(rows_b, gsb)
            fire_s(2 * p + 1, rows_b, wsb)
            drain_r(rows_a, wsa)
            fire_r(2 * p + 2, rows_a, gsa)
            return carry

        lax.fori_loop(0, PS, body, 0)
        drain_r(rows_a, gsa)
        fire_s(CPT - 1, rows_a, wsa)
        drain_r(rows_b, wsb)
        drain_r(rows_a, wsa)
        plsc.subcore_barrier()
        pltpu.sync_copy(acc.at[pl.ds(sid * NPT8, NPT8)],
                        out.at[pl.ds(sid * NPT8, NPT8)])

        @pl.when(sid == NS - 1)
        def _():
            pltpu.sync_copy(acc.at[pl.ds(NS * NPT8, NREM)],
                            out.at[pl.ds(NS * NPT8, NREM)])

    @pl.when(cid == 0)
    def _():
        run(s_hbm, aggs_hbm)

    @pl.when(cid == 1)
    def _():
        run(r_hbm, aggr_hbm)


_sc_scatter = pl.kernel(
    _sc_scatter_body,
    out_type=[jax.ShapeDtypeStruct((NN, D), jnp.float32),
              jax.ShapeDtypeStruct((NN, D), jnp.float32)],
    mesh=_SC_MESH,
    scratch_types=[pltpu.VMEM((CPT, CHUNK), jnp.int32),
                   pltpu.VMEM((CHUNK, D), jnp.float32),
                   pltpu.VMEM((CHUNK, D), jnp.float32),
                   pltpu.SemaphoreType.DMA,
                   pltpu.SemaphoreType.DMA,
                   pltpu.SemaphoreType.DMA,
                   pltpu.SemaphoreType.DMA,
                   pltpu.VMEM_SHARED((NN, D), jnp.float32)],
)


def _ln_relu(h, scale, bias):
    mean = jnp.mean(h, axis=-1, keepdims=True)
    var = jnp.mean(jnp.square(h - mean), axis=-1, keepdims=True)
    return jnp.maximum((h - mean) * lax.rsqrt(var + EPS) * scale + bias, 0.0)


def _dot(a, b):
    return jnp.dot(a, b, preferred_element_type=jnp.float32)


BE = 2000
GE = NE // BE
BN = 2000
GN = NN // BN

_blk = lambda r: pl.BlockSpec((r, D), lambda i: (i, 0))
_full = lambda s: pl.BlockSpec(s, lambda i: (0, 0))


def _embed_nodes_body(n_ref, Wn, bn, Ws, Wr, x_out, xs_out, xr_out):
    x = _dot(n_ref[...], Wn[...]) + bn[...]
    x_out[...] = x
    xs_out[...] = _dot(x, Ws[...])
    xr_out[...] = _dot(x, Wr[...])


def _embed_nodes(nodes, Wn, bn, Ws, Wr):
    return pl.pallas_call(
        _embed_nodes_body,
        grid=(GN,),
        in_specs=[_blk(BN), _full((D, D)), _full((1, D)), _full((D, D)), _full((D, D))],
        out_specs=[_blk(BN), _blk(BN), _blk(BN)],
        out_shape=[jax.ShapeDtypeStruct((NN, D), jnp.float32)] * 3,
    )(nodes, Wn, bn, Ws, Wr)


def _embed_edges_body(e_ref, We, be, out_ref):
    out_ref[...] = _dot(e_ref[...], We[...]) + be[...]


def _embed_edges(edges, We, be):
    de = edges.shape[-1]
    return pl.pallas_call(
        _embed_edges_body,
        grid=(GE,),
        in_specs=[pl.BlockSpec((BE, de), lambda i: (i, 0)),
                  _full((de, D)), _full((1, D))],
        out_specs=_blk(BE),
        out_shape=jax.ShapeDtypeStruct((NE, D), jnp.float32),
    )(edges, We, be)


def _edge_body(e_ref, gs_ref, gr_ref, g_ref, We, Wg, b, scale, bias,
               out_ref, agg_ref):
    c = _dot(g_ref[...], Wg[...]) + b[...]
    h = _dot(e_ref[...], We[...]) + gs_ref[...] + gr_ref[...] + c
    en = _ln_relu(h, scale[...], bias[...])
    out_ref[...] = en

    @pl.when(pl.program_id(0) == 0)
    def _():
        agg_ref[...] = jnp.zeros_like(agg_ref)

    agg_ref[...] += jnp.sum(en, axis=0, keepdims=True)


def _edge_update(e, gs, gr, g, We, Wg, b, scale, bias):
    return pl.pallas_call(
        _edge_body,
        grid=(GE,),
        in_specs=[_blk(BE), _blk(BE), _blk(BE), _full((1, D)),
                  _full((D, D)), _full((D, D)),
                  _full((1, D)), _full((1, D)), _full((1, D))],
        out_specs=[_blk(BE), _full((1, D))],
        out_shape=[jax.ShapeDtypeStruct((NE, D), jnp.float32),
                   jax.ShapeDtypeStruct((1, D), jnp.float32)],
        compiler_params=pltpu.CompilerParams(dimension_semantics=("arbitrary",)),
    )(e, gs, gr, g, We, Wg, b, scale, bias)


def _node_body(x_ref, as_ref, ar_ref, g_ref, Vx, Vas, Var, Vg, b, scale, bias,
               Wsn, Wrn, x_out, agg_ref, xs_out, xr_out):
    c = _dot(g_ref[...], Vg[...]) + b[...]
    h = (_dot(x_ref[...], Vx[...]) + _dot(as_ref[...], Vas[...])
         + _dot(ar_ref[...], Var[...]) + c)
    xn = _ln_relu(h, scale[...], bias[...])
    x_out[...] = xn

    @pl.when(pl.program_id(0) == 0)
    def _():
        agg_ref[...] = jnp.zeros_like(agg_ref)

    agg_ref[...] += jnp.sum(xn, axis=0, keepdims=True)
    xs_out[...] = _dot(xn, Wsn[...])
    xr_out[...] = _dot(xn, Wrn[...])


def _node_update(x, aggs, aggr, g, Vx, Vas, Var, Vg, b, scale, bias, Wsn, Wrn):
    return pl.pallas_call(
        _node_body,
        grid=(GN,),
        in_specs=[_blk(BN), _blk(BN), _blk(BN), _full((1, D)),
                  _full((D, D)), _full((D, D)), _full((D, D)), _full((D, D)),
                  _full((1, D)), _full((1, D)), _full((1, D)),
                  _full((D, D)), _full((D, D))],
        out_specs=[_blk(BN), _full((1, D)), _blk(BN), _blk(BN)],
        out_shape=[jax.ShapeDtypeStruct((NN, D), jnp.float32),
                   jax.ShapeDtypeStruct((1, D), jnp.float32),
                   jax.ShapeDtypeStruct((NN, D), jnp.float32),
                   jax.ShapeDtypeStruct((NN, D), jnp.float32)],
        compiler_params=pltpu.CompilerParams(dimension_semantics=("arbitrary",)),
    )(x, aggs, aggr, g, Vx, Vas, Var, Vg, b, scale, bias, Wsn, Wrn)


def _node_last_body(x_ref, as_ref, ar_ref, g_ref, Vx, Vas, Var, Vg, b, scale,
                    bias, x_out, agg_ref):
    c = _dot(g_ref[...], Vg[...]) + b[...]
    h = (_dot(x_ref[...], Vx[...]) + _dot(as_ref[...], Vas[...])
         + _dot(ar_ref[...], Var[...]) + c)
    xn = _ln_relu(h, scale[...], bias[...])
    x_out[...] = xn

    @pl.when(pl.program_id(0) == 0)
    def _():
        agg_ref[...] = jnp.zeros_like(agg_ref)

    agg_ref[...] += jnp.sum(xn, axis=0, keepdims=True)


def _node_update_last(x, aggs, aggr, g, Vx, Vas, Var, Vg, b, scale, bias):
    return pl.pallas_call(
        _node_last_body,
        grid=(GN,),
        in_specs=[_blk(BN), _blk(BN), _blk(BN), _full((1, D)),
                  _full((D, D)), _full((D, D)), _full((D, D)), _full((D, D)),
                  _full((1, D)), _full((1, D)), _full((1, D))],
        out_specs=[_blk(BN), _full((1, D))],
        out_shape=[jax.ShapeDtypeStruct((NN, D), jnp.float32),
                   jax.ShapeDtypeStruct((1, D), jnp.float32)],
        compiler_params=pltpu.CompilerParams(dimension_semantics=("arbitrary",)),
    )(x, aggs, aggr, g, Vx, Vas, Var, Vg, b, scale, bias)


def _global_body(na_ref, ea_ref, g_ref, W_ref, b, scale, bias, out_ref):
    W = W_ref[...]
    h = (_dot(na_ref[...], W[0:D]) + _dot(ea_ref[...], W[D:2 * D])
         + _dot(g_ref[...], W[2 * D:3 * D]) + b[...])
    out_ref[...] = _ln_relu(h, scale[...], bias[...])


def _global_update(na, ea, g, W, b, scale, bias):
    return pl.pallas_call(
        _global_body,
        out_shape=jax.ShapeDtypeStruct((1, D), jnp.float32),
    )(na, ea, g, W, b, scale, bias)


def _decode_body(g_ref, Wd, bd, out_ref):
    out_ref[...] = _dot(g_ref[...], Wd[...]) + bd[...]


def _decode(g, Wd, bd):
    return pl.pallas_call(
        _decode_body,
        out_shape=jax.ShapeDtypeStruct((1, D), jnp.float32),
    )(g, Wd, bd)


def kernel(nodes, edges, senders, receivers, node_embed, edge_embed,
           step_params, decoder, train=False):
    s2d = senders.reshape(NS, CPT, CHUNK)
    r2d = receivers.reshape(NS, CPT, CHUNK)
    zeros = jnp.zeros((NN, D), jnp.float32)
    g = jnp.zeros((1, D), jnp.float32)

    def row(v):
        return v.reshape(1, D)

    W0 = step_params[0]['edge'][0]
    x, xs, xr = _embed_nodes(nodes, node_embed[0], row(node_embed[1]),
                             W0[D:2 * D], W0[2 * D:3 * D])
    e = _embed_edges(edges, edge_embed[0], row(edge_embed[1]))

    for t in range(len(step_params)):
        ep = step_params[t]['edge']
        npar = step_params[t]['node']
        gp = step_params[t]['global']
        W = ep[0]
        gs, gr = _sc_gather(xs, xr, s2d, r2d)
        e, e_agg = _edge_update(e, gs, gr, g, W[0:D], W[3 * D:4 * D],
                                row(ep[1]), row(ep[2]), row(ep[3]))
        agg_s, agg_r = _sc_scatter(e, s2d, r2d, zeros)
        V = npar[0]
        if t + 1 < len(step_params):
            Wn = step_params[t + 1]['edge'][0]
            x, n_agg, xs, xr = _node_update(
                x, agg_s, agg_r, g, V[0:D], V[D:2 * D], V[2 * D:3 * D],
                V[3 * D:4 * D], row(npar[1]), row(npar[2]), row(npar[3]),
                Wn[D:2 * D], Wn[2 * D:3 * D])
        else:
            x, n_agg = _node_update_last(
                x, agg_s, agg_r, g, V[0:D], V[D:2 * D], V[2 * D:3 * D],
                V[3 * D:4 * D], row(npar[1]), row(npar[2]), row(npar[3]))
        g = _global_update(n_agg, e_agg, g, gp[0], row(gp[1]), row(gp[2]),
                           row(gp[3]))

    return _decode(g, decoder[0], row(decoder[1]))


# speedup vs baseline: 4.0975x; 1.0055x over previous
import jax
import jax.numpy as jnp
from jax import lax
from jax.experimental import pallas as pl
from jax.experimental.pallas import tpu as pltpu
from jax.experimental.pallas import tpu_sc as plsc

NN = 10000
NE = 160000
D = 128
EPS = 1e-6

NC, NS = 2, 16
CHUNK = 80
CPT = NE // (NS * CHUNK)
NPT8 = (NN // NS) // 8 * 8
NREM = NN - NS * NPT8

_SC_MESH = plsc.VectorSubcoreMesh(core_axis_name="c", subcore_axis_name="s",
                                  num_cores=NC, num_subcores=NS)


K = 5
KC = K * CHUNK
G = CPT // K
P = (G - 1) // 2
PS = (CPT - 1) // 2


DW = D // 2


def _sc_gather_body(xs_hbm, xr_hbm, s_hbm, r_hbm, gs_hbm, gr_hbm,
                    idx_v, rows_a, rows_b, gsa, gsb, wsa, wsb):
    cid = lax.axis_index("c")
    sid = lax.axis_index("s")

    def run(tab, idx3d, out):
        pltpu.sync_copy(idx3d.at[sid], idx_v)

        def fire_g(g, buf, sem):
            for k in range(K):
                pltpu.async_copy(tab.at[idx_v.at[g * K + k]],
                                 buf.at[pl.ds(k * CHUNK, CHUNK)], sem)

        def drain_g(buf, sem):
            pltpu.make_async_copy(tab.at[pl.ds(0, KC)], buf, sem).wait()

        def fire_w(g, buf, sem):
            pltpu.async_copy(buf, out.at[pl.ds((sid * G + g) * KC, KC)], sem)

        def drain_w(buf, sem):
            pltpu.make_async_copy(buf, out.at[pl.ds(0, KC)], sem).wait()

        fire_g(0, rows_a, gsa)

        def body(p, carry):
            drain_g(rows_a, gsa)
            fire_w(2 * p, rows_a, wsa)

            @pl.when(p > 0)
            def _():
                drain_w(rows_b, wsb)

            fire_g(2 * p + 1, rows_b, gsb)
            drain_g(rows_b, gsb)
            fire_w(2 * p + 1, rows_b, wsb)
            drain_w(rows_a, wsa)
            fire_g(2 * p + 2, rows_a, gsa)
            return carry

        lax.fori_loop(0, P, body, 0)
        drain_g(rows_a, gsa)
        fire_w(G - 1, rows_a, wsa)
        drain_w(rows_b, wsb)
        drain_w(rows_a, wsa)

    @pl.when(cid == 0)
    def _():
        run(xs_hbm, s_hbm, gs_hbm)

    @pl.when(cid == 1)
    def _():
        run(xr_hbm, r_hbm, gr_hbm)


_sc_gather = pl.kernel(
    _sc_gather_body,
    out_type=[jax.ShapeDtypeStruct((NE, D), jnp.float32),
              jax.ShapeDtypeStruct((NE, D), jnp.float32)],
    mesh=_SC_MESH,
    scratch_types=[pltpu.VMEM((CPT, CHUNK), jnp.int32),
                   pltpu.VMEM((KC, D), jnp.float32),
                   pltpu.VMEM((KC, D), jnp.float32),
                   pltpu.SemaphoreType.DMA,
                   pltpu.SemaphoreType.DMA,
                   pltpu.SemaphoreType.DMA,
                   pltpu.SemaphoreType.DMA],
)


def _sc_scatter_body(e_hbm, s_hbm, r_hbm, zero_hbm, aggs_hbm, aggr_hbm,
                     idx_v, rows_a, rows_b, gsa, gsb, wsa, wsb, acc):
    cid = lax.axis_index("c")
    sid = lax.axis_index("s")
    pltpu.sync_copy(zero_hbm.at[pl.ds(sid * NPT8, NPT8)],
                    acc.at[pl.ds(sid * NPT8, NPT8)])

    @pl.when(sid == NS - 1)
    def _():
        pltpu.sync_copy(zero_hbm.at[pl.ds(NS * NPT8, NREM)],
                        acc.at[pl.ds(NS * NPT8, NREM)])

    plsc.subcore_barrier()

    def run(idx3d, out):
        pltpu.sync_copy(idx3d.at[sid], idx_v)

        def fire_r(j, buf, sem):
            pltpu.async_copy(e_hbm.at[pl.ds((sid * CPT + j) * CHUNK, CHUNK)],
                             buf, sem)

        def drain_r(buf, sem):
            pltpu.make_async_copy(e_hbm.at[pl.ds(0, CHUNK)], buf, sem).wait()

        def fire_s(j, buf, sem):
            pltpu.async_copy(buf, acc.at[idx_v.at[j]], sem, add=True)

        fire_r(0, rows_a, gsa)

        def body(p, carry):
            drain_r(rows_a, gsa)
            fire_s(2 * p, rows_a, wsa)

            @pl.when(p > 0)
            def _():
                drain_r(rows_b, wsb)

            fire_r(2 * p + 1, rows_b, gsb)
            drain_r(rows_b, gsb)
            fire_s(2 * p + 1, rows_b, wsb)
            drain_r(rows_a, wsa)
            fire_r(2 * p + 2, rows_a, gsa)
            return carry

        lax.fori_loop(0, PS, body, 0)
        drain_r(rows_a, gsa)
        fire_s(CPT - 1, rows_a, wsa)
        drain_r(rows_b, wsb)
        drain_r(rows_a, wsa)
        plsc.subcore_barrier()
        pltpu.sync_copy(acc.at[pl.ds(sid * NPT8, NPT8)],
                        out.at[pl.ds(sid * NPT8, NPT8)])

        @pl.when(sid == NS - 1)
        def _():
            pltpu.sync_copy(acc.at[pl.ds(NS * NPT8, NREM)],
                            out.at[pl.ds(NS * NPT8, NREM)])

    @pl.when(cid == 0)
    def _():
        run(s_hbm, aggs_hbm)

    @pl.when(cid == 1)
    def _():
        run(r_hbm, aggr_hbm)


_sc_scatter = pl.kernel(
    _sc_scatter_body,
    out_type=[jax.ShapeDtypeStruct((NN, D), jnp.float32),
              jax.ShapeDtypeStruct((NN, D), jnp.float32)],
    mesh=_SC_MESH,
    scratch_types=[pltpu.VMEM((CPT, CHUNK), jnp.int32),
                   pltpu.VMEM((CHUNK, D), jnp.float32),
                   pltpu.VMEM((CHUNK, D), jnp.float32),
                   pltpu.SemaphoreType.DMA,
                   pltpu.SemaphoreType.DMA,
                   pltpu.SemaphoreType.DMA,
                   pltpu.SemaphoreType.DMA,
                   pltpu.VMEM_SHARED((NN, D), jnp.float32)],
)


def _ln_relu(h, scale, bias):
    mean = jnp.mean(h, axis=-1, keepdims=True)
    var = jnp.mean(jnp.square(h - mean), axis=-1, keepdims=True)
    return jnp.maximum((h - mean) * lax.rsqrt(var + EPS) * scale + bias, 0.0)


def _dot(a, b):
    return jnp.dot(a, b, preferred_element_type=jnp.float32)


BE = 2000
GE = NE // BE
BN = 2000
GN = NN // BN

_blk = lambda r: pl.BlockSpec((r, D), lambda i: (i, 0))
_full = lambda s: pl.BlockSpec(s, lambda i: (0, 0))


def _embed_nodes_body(n_ref, Wn, bn, Ws, Wr, x_out, xs_out, xr_out):
    x = _dot(n_ref[...], Wn[...]) + bn[...]
    x_out[...] = x
    xs_out[...] = _dot(x, Ws[...])
    xr_out[...] = _dot(x, Wr[...])


def _embed_nodes(nodes, Wn, bn, Ws, Wr):
    return pl.pallas_call(
        _embed_nodes_body,
        grid=(GN,),
        in_specs=[_blk(BN), _full((D, D)), _full((1, D)), _full((D, D)), _full((D, D))],
        out_specs=[_blk(BN), _blk(BN), _blk(BN)],
        out_shape=[jax.ShapeDtypeStruct((NN, D), jnp.float32)] * 3,
    )(nodes, Wn, bn, Ws, Wr)


def _embed_edges_body(e_ref, We, be, out_ref):
    out_ref[...] = _dot(e_ref[...], We[...]) + be[...]


def _embed_edges(edges, We, be):
    de = edges.shape[-1]
    return pl.pallas_call(
        _embed_edges_body,
        grid=(GE,),
        in_specs=[pl.BlockSpec((BE, de), lambda i: (i, 0)),
                  _full((de, D)), _full((1, D))],
        out_specs=_blk(BE),
        out_shape=jax.ShapeDtypeStruct((NE, D), jnp.float32),
    )(edges, We, be)


def _edge_body(e_ref, gs_ref, gr_ref, g_ref, We, Wg, b, scale, bias,
               out_ref, agg_ref):
    c = _dot(g_ref[...], Wg[...]) + b[...]
    eb = e_ref[...].astype(jnp.bfloat16)
    h = _dot(eb, We[...]) + gs_ref[...] + gr_ref[...] + c
    en = _ln_relu(h, scale[...], bias[...])
    out_ref[...] = en

    @pl.when(pl.program_id(0) == 0)
    def _():
        agg_ref[...] = jnp.zeros_like(agg_ref)

    agg_ref[...] += jnp.sum(en, axis=0, keepdims=True)


def _edge_update(e, gs, gr, g, We, Wg, b, scale, bias):
    return pl.pallas_call(
        _edge_body,
        grid=(GE,),
        in_specs=[_blk(BE), _blk(BE), _blk(BE), _full((1, D)),
                  _full((D, D)), _full((D, D)),
                  _full((1, D)), _full((1, D)), _full((1, D))],
        out_specs=[_blk(BE), _full((1, D))],
        out_shape=[jax.ShapeDtypeStruct((NE, D), jnp.float32),
                   jax.ShapeDtypeStruct((1, D), jnp.float32)],
        compiler_params=pltpu.CompilerParams(dimension_semantics=("arbitrary",)),
    )(e, gs, gr, g, We, Wg, b, scale, bias)


def _node_body(x_ref, as_ref, ar_ref, g_ref, Vx, Vas, Var, Vg, b, scale, bias,
               Wsn, Wrn, x_out, agg_ref, xs_out, xr_out):
    c = _dot(g_ref[...], Vg[...]) + b[...]
    h = (_dot(x_ref[...].astype(jnp.bfloat16), Vx[...])
         + _dot(as_ref[...].astype(jnp.bfloat16), Vas[...])
         + _dot(ar_ref[...].astype(jnp.bfloat16), Var[...]) + c)
    xn = _ln_relu(h, scale[...], bias[...])
    x_out[...] = xn

    @pl.when(pl.program_id(0) == 0)
    def _():
        agg_ref[...] = jnp.zeros_like(agg_ref)

    agg_ref[...] += jnp.sum(xn, axis=0, keepdims=True)
    xnb = xn.astype(jnp.bfloat16)
    xs_out[...] = _dot(xnb, Wsn[...])
    xr_out[...] = _dot(xnb, Wrn[...])


def _node_update(x, aggs, aggr, g, Vx, Vas, Var, Vg, b, scale, bias, Wsn, Wrn):
    return pl.pallas_call(
        _node_body,
        grid=(GN,),
        in_specs=[_blk(BN), _blk(BN), _blk(BN), _full((1, D)),
                  _full((D, D)), _full((D, D)), _full((D, D)), _full((D, D)),
                  _full((1, D)), _full((1, D)), _full((1, D)),
                  _full((D, D)), _full((D, D))],
        out_specs=[_blk(BN), _full((1, D)), _blk(BN), _blk(BN)],
        out_shape=[jax.ShapeDtypeStruct((NN, D), jnp.float32),
                   jax.ShapeDtypeStruct((1, D), jnp.float32),
                   jax.ShapeDtypeStruct((NN, D), jnp.float32),
                   jax.ShapeDtypeStruct((NN, D), jnp.float32)],
        compiler_params=pltpu.CompilerParams(dimension_semantics=("arbitrary",)),
    )(x, aggs, aggr, g, Vx, Vas, Var, Vg, b, scale, bias, Wsn, Wrn)


def _node_last_body(x_ref, as_ref, ar_ref, g_ref, Vx, Vas, Var, Vg, b, scale,
                    bias, x_out, agg_ref):
    c = _dot(g_ref[...], Vg[...]) + b[...]
    h = (_dot(x_ref[...].astype(jnp.bfloat16), Vx[...])
         + _dot(as_ref[...].astype(jnp.bfloat16), Vas[...])
         + _dot(ar_ref[...].astype(jnp.bfloat16), Var[...]) + c)
    xn = _ln_relu(h, scale[...], bias[...])
    x_out[...] = xn

    @pl.when(pl.program_id(0) == 0)
    def _():
        agg_ref[...] = jnp.zeros_like(agg_ref)

    agg_ref[...] += jnp.sum(xn, axis=0, keepdims=True)


def _node_update_last(x, aggs, aggr, g, Vx, Vas, Var, Vg, b, scale, bias):
    return pl.pallas_call(
        _node_last_body,
        grid=(GN,),
        in_specs=[_blk(BN), _blk(BN), _blk(BN), _full((1, D)),
                  _full((D, D)), _full((D, D)), _full((D, D)), _full((D, D)),
                  _full((1, D)), _full((1, D)), _full((1, D))],
        out_specs=[_blk(BN), _full((1, D))],
        out_shape=[jax.ShapeDtypeStruct((NN, D), jnp.float32),
                   jax.ShapeDtypeStruct((1, D), jnp.float32)],
        compiler_params=pltpu.CompilerParams(dimension_semantics=("arbitrary",)),
    )(x, aggs, aggr, g, Vx, Vas, Var, Vg, b, scale, bias)


def _global_body(na_ref, ea_ref, g_ref, W_ref, b, scale, bias, out_ref):
    W = W_ref[...]
    h = (_dot(na_ref[...], W[0:D]) + _dot(ea_ref[...], W[D:2 * D])
         + _dot(g_ref[...], W[2 * D:3 * D]) + b[...])
    out_ref[...] = _ln_relu(h, scale[...], bias[...])


def _global_update(na, ea, g, W, b, scale, bias):
    return pl.pallas_call(
        _global_body,
        out_shape=jax.ShapeDtypeStruct((1, D), jnp.float32),
    )(na, ea, g, W, b, scale, bias)


def _decode_body(g_ref, Wd, bd, out_ref):
    out_ref[...] = _dot(g_ref[...], Wd[...]) + bd[...]


def _decode(g, Wd, bd):
    return pl.pallas_call(
        _decode_body,
        out_shape=jax.ShapeDtypeStruct((1, D), jnp.float32),
    )(g, Wd, bd)


def kernel(nodes, edges, senders, receivers, node_embed, edge_embed,
           step_params, decoder, train=False):
    s2d = senders.reshape(NS, CPT, CHUNK)
    r2d = receivers.reshape(NS, CPT, CHUNK)
    zeros = jnp.zeros((NN, D), jnp.float32)
    g = jnp.zeros((1, D), jnp.float32)

    def row(v):
        return v.reshape(1, D)

    W0 = step_params[0]['edge'][0]
    x, xs, xr = _embed_nodes(nodes, node_embed[0], row(node_embed[1]),
                             W0[D:2 * D], W0[2 * D:3 * D])
    e = _embed_edges(edges, edge_embed[0], row(edge_embed[1]))

    for t in range(len(step_params)):
        ep = step_params[t]['edge']
        npar = step_params[t]['node']
        gp = step_params[t]['global']
        W = ep[0]
        bf = lambda a: a.astype(jnp.bfloat16)
        gs, gr = _sc_gather(xs, xr, s2d, r2d)
        e, e_agg = _edge_update(e, gs, gr, g, bf(W[0:D]), W[3 * D:4 * D],
                                row(ep[1]), row(ep[2]), row(ep[3]))
        agg_s, agg_r = _sc_scatter(e, s2d, r2d, zeros)
        V = npar[0]
        if t + 1 < len(step_params):
            Wn = step_params[t + 1]['edge'][0]
            x, n_agg, xs, xr = _node_update(
                x, agg_s, agg_r, g, bf(V[0:D]), bf(V[D:2 * D]),
                bf(V[2 * D:3 * D]), V[3 * D:4 * D],
                row(npar[1]), row(npar[2]), row(npar[3]),
                bf(Wn[D:2 * D]), bf(Wn[2 * D:3 * D]))
        else:
            x, n_agg = _node_update_last(
                x, agg_s, agg_r, g, bf(V[0:D]), bf(V[D:2 * D]),
                bf(V[2 * D:3 * D]), V[3 * D:4 * D],
                row(npar[1]), row(npar[2]), row(npar[3]))
        g = _global_update(n_agg, e_agg, g, gp[0], row(gp[1]), row(gp[2]),
                           row(gp[3]))

    return _decode(g, decoder[0], row(decoder[1]))


# speedup vs baseline: 4.3467x; 1.0608x over previous
import jax
import jax.numpy as jnp
from jax import lax
from jax.experimental import pallas as pl
from jax.experimental.pallas import tpu as pltpu
from jax.experimental.pallas import tpu_sc as plsc

NN = 10000
NE = 160000
D = 128
EPS = 1e-6

NC, NS = 2, 16
CHUNK = 80
EA = 96000
EB = NE - EA
K = 5
KC = K * CHUNK
NPT8 = (NN // NS) // 8 * 8
NREM = NN - NS * NPT8

_SC_MESH = plsc.VectorSubcoreMesh(core_axis_name="c", subcore_axis_name="s",
                                  num_cores=NC, num_subcores=NS)


def _pipeline(n_groups, fire_in, drain_in, fire_out, drain_out):
    fire_in(0, 0)

    def body(p, carry):
        drain_in(0)
        fire_out(2 * p, 0)

        @pl.when(p > 0)
        def _():
            drain_out(1)

        fire_in(2 * p + 1, 1)
        drain_in(1)
        fire_out(2 * p + 1, 1)
        drain_out(0)
        fire_in(2 * p + 2, 0)
        return carry

    lax.fori_loop(0, (n_groups - 1) // 2, body, 0)
    if n_groups % 2:
        drain_in(0)
        fire_out(n_groups - 1, 0)
        drain_out(1)
        drain_out(0)
    else:
        drain_in(0)
        fire_out(n_groups - 2, 0)
        drain_out(1)
        fire_in(n_groups - 1, 1)
        drain_in(1)
        fire_out(n_groups - 1, 1)
        drain_out(0)
        drain_out(1)


def _make_sc_gather(n_edges):
    cpt = n_edges // (NS * CHUNK)
    ng = cpt // K

    def body(xs_hbm, xr_hbm, s_hbm, r_hbm, gs_hbm, gr_hbm,
             idx_v, rows_a, rows_b, sin_a, sin_b, sout_a, sout_b):
        cid = lax.axis_index("c")
        sid = lax.axis_index("s")
        rows = (rows_a, rows_b)
        sin = (sin_a, sin_b)
        sout = (sout_a, sout_b)

        def run(tab, idx3d, out):
            pltpu.sync_copy(idx3d.at[sid], idx_v)

            def fire_in(g, i):
                for k in range(K):
                    pltpu.async_copy(tab.at[idx_v.at[g * K + k]],
                                     rows[i].at[pl.ds(k * CHUNK, CHUNK)],
                                     sin[i])

            def drain_in(i):
                pltpu.make_async_copy(tab.at[pl.ds(0, KC)], rows[i],
                                      sin[i]).wait()

            def fire_out(g, i):
                pltpu.async_copy(rows[i],
                                 out.at[pl.ds((sid * ng + g) * KC, KC)],
                                 sout[i])

            def drain_out(i):
                pltpu.make_async_copy(rows[i], out.at[pl.ds(0, KC)],
                                      sout[i]).wait()

            _pipeline(ng, fire_in, drain_in, fire_out, drain_out)

        @pl.when(cid == 0)
        def _():
            run(xs_hbm, s_hbm, gs_hbm)

        @pl.when(cid == 1)
        def _():
            run(xr_hbm, r_hbm, gr_hbm)

    return pl.kernel(
        body,
        out_type=[jax.ShapeDtypeStruct((n_edges, D), jnp.float32),
                  jax.ShapeDtypeStruct((n_edges, D), jnp.float32)],
        mesh=_SC_MESH,
        scratch_types=[pltpu.VMEM((cpt, CHUNK), jnp.int32),
                       pltpu.VMEM((KC, D), jnp.float32),
                       pltpu.VMEM((KC, D), jnp.float32),
                       pltpu.SemaphoreType.DMA,
                       pltpu.SemaphoreType.DMA,
                       pltpu.SemaphoreType.DMA,
                       pltpu.SemaphoreType.DMA],
    )


def _make_sc_scatter(n_edges):
    cpt = n_edges // (NS * CHUNK)

    def body(e_hbm, s_hbm, r_hbm, zero_hbm, aggs_hbm, aggr_hbm,
             idx_v, rows_a, rows_b, sin_a, sin_b, sout_a, sout_b, acc):
        cid = lax.axis_index("c")
        sid = lax.axis_index("s")
        rows = (rows_a, rows_b)
        sin = (sin_a, sin_b)
        sout = (sout_a, sout_b)
        pltpu.sync_copy(zero_hbm.at[pl.ds(sid * NPT8, NPT8)],
                        acc.at[pl.ds(sid * NPT8, NPT8)])

        @pl.when(sid == NS - 1)
        def _():
            pltpu.sync_copy(zero_hbm.at[pl.ds(NS * NPT8, NREM)],
                            acc.at[pl.ds(NS * NPT8, NREM)])

        plsc.subcore_barrier()

        def run(idx3d, out):
            pltpu.sync_copy(idx3d.at[sid], idx_v)

            def fire_in(j, i):
                pltpu.async_copy(
                    e_hbm.at[pl.ds((sid * cpt + j) * CHUNK, CHUNK)],
                    rows[i], sin[i])

            def drain_in(i):
                pltpu.make_async_copy(e_hbm.at[pl.ds(0, CHUNK)], rows[i],
                                      sin[i]).wait()

            def fire_out(j, i):
                pltpu.async_copy(rows[i], acc.at[idx_v.at[j]], sout[i],
                                 add=True)

            def drain_out(i):
                pltpu.make_async_copy(e_hbm.at[pl.ds(0, CHUNK)], rows[i],
                                      sout[i]).wait()

            _pipeline(cpt, fire_in, drain_in, fire_out, drain_out)
            plsc.subcore_barrier()
            pltpu.sync_copy(acc.at[pl.ds(sid * NPT8, NPT8)],
                            out.at[pl.ds(sid * NPT8, NPT8)])

            @pl.when(sid == NS - 1)
            def _():
                pltpu.sync_copy(acc.at[pl.ds(NS * NPT8, NREM)],
                                out.at[pl.ds(NS * NPT8, NREM)])

        @pl.when(cid == 0)
        def _():
            run(s_hbm, aggs_hbm)

        @pl.when(cid == 1)
        def _():
            run(r_hbm, aggr_hbm)

    return pl.kernel(
        body,
        out_type=[jax.ShapeDtypeStruct((NN, D), jnp.float32),
                  jax.ShapeDtypeStruct((NN, D), jnp.float32)],
        mesh=_SC_MESH,
        scratch_types=[pltpu.VMEM((cpt, CHUNK), jnp.int32),
                       pltpu.VMEM((CHUNK, D), jnp.float32),
                       pltpu.VMEM((CHUNK, D), jnp.float32),
                       pltpu.SemaphoreType.DMA,
                       pltpu.SemaphoreType.DMA,
                       pltpu.SemaphoreType.DMA,
                       pltpu.SemaphoreType.DMA,
                       pltpu.VMEM_SHARED((NN, D), jnp.float32)],
    )


_sc_gather_a = _make_sc_gather(EA)
_sc_gather_b = _make_sc_gather(EB)
_sc_scatter_a = _make_sc_scatter(EA)
_sc_scatter_b = _make_sc_scatter(EB)


def _ln_relu(h, scale, bias):
    mean = jnp.mean(h, axis=-1, keepdims=True)
    var = jnp.mean(jnp.square(h - mean), axis=-1, keepdims=True)
    return jnp.maximum((h - mean) * lax.rsqrt(var + EPS) * scale + bias, 0.0)


def _dot(a, b):
    return jnp.dot(a, b, preferred_element_type=jnp.float32)


BE = 2000
BN = 2000
GN = NN // BN

_blk = lambda r: pl.BlockSpec((r, D), lambda i: (i, 0))
_full = lambda s: pl.BlockSpec(s, lambda i: (0, 0))


def _embed_nodes_body(n_ref, Wn, bn, Ws, Wr, x_out, xs_out, xr_out):
    x = _dot(n_ref[...], Wn[...]) + bn[...]
    x_out[...] = x
    xs_out[...] = _dot(x, Ws[...])
    xr_out[...] = _dot(x, Wr[...])


def _embed_nodes(nodes, Wn, bn, Ws, Wr):
    return pl.pallas_call(
        _embed_nodes_body,
        grid=(GN,),
        in_specs=[_blk(BN), _full((D, D)), _full((1, D)), _full((D, D)), _full((D, D))],
        out_specs=[_blk(BN), _blk(BN), _blk(BN)],
        out_shape=[jax.ShapeDtypeStruct((NN, D), jnp.float32)] * 3,
    )(nodes, Wn, bn, Ws, Wr)


def _embed_edges_body(e_ref, We, be, out_ref):
    out_ref[...] = _dot(e_ref[...], We[...]) + be[...]


def _embed_edges(edges, We, be):
    rows, de = edges.shape
    return pl.pallas_call(
        _embed_edges_body,
        grid=(rows // BE,),
        in_specs=[pl.BlockSpec((BE, de), lambda i: (i, 0)),
                  _full((de, D)), _full((1, D))],
        out_specs=_blk(BE),
        out_shape=jax.ShapeDtypeStruct((rows, D), jnp.float32),
    )(edges, We, be)


def _edge_body(e_ref, gs_ref, gr_ref, g_ref, We, Wg, b, scale, bias,
               out_ref, agg_ref):
    c = _dot(g_ref[...], Wg[...]) + b[...]
    h = _dot(e_ref[...], We[...]) + gs_ref[...] + gr_ref[...] + c
    en = _ln_relu(h, scale[...], bias[...])
    out_ref[...] = en

    @pl.when(pl.program_id(0) == 0)
    def _():
        agg_ref[...] = jnp.zeros_like(agg_ref)

    agg_ref[...] += jnp.sum(en, axis=0, keepdims=True)


def _edge_update(e, gs, gr, g, We, Wg, b, scale, bias):
    rows = e.shape[0]
    return pl.pallas_call(
        _edge_body,
        grid=(rows // BE,),
        in_specs=[_blk(BE), _blk(BE), _blk(BE), _full((1, D)),
                  _full((D, D)), _full((D, D)),
                  _full((1, D)), _full((1, D)), _full((1, D))],
        out_specs=[_blk(BE), _full((1, D))],
        out_shape=[jax.ShapeDtypeStruct((rows, D), jnp.float32),
                   jax.ShapeDtypeStruct((1, D), jnp.float32)],
        compiler_params=pltpu.CompilerParams(dimension_semantics=("arbitrary",)),
    )(e, gs, gr, g, We, Wg, b, scale, bias)


def _node_body(x_ref, asa_ref, asb_ref, ara_ref, arb_ref, g_ref,
               Vx, Vas, Var, Vg, b, scale, bias,
               Wsn, Wrn, x_out, agg_ref, xs_out, xr_out):
    c = _dot(g_ref[...], Vg[...]) + b[...]
    h = (_dot(x_ref[...], Vx[...])
         + _dot(asa_ref[...] + asb_ref[...], Vas[...])
         + _dot(ara_ref[...] + arb_ref[...], Var[...]) + c)
    xn = _ln_relu(h, scale[...], bias[...])
    x_out[...] = xn

    @pl.when(pl.program_id(0) == 0)
    def _():
        agg_ref[...] = jnp.zeros_like(agg_ref)

    agg_ref[...] += jnp.sum(xn, axis=0, keepdims=True)
    xs_out[...] = _dot(xn, Wsn[...])
    xr_out[...] = _dot(xn, Wrn[...])


def _node_update(x, asa, asb, ara, arb, g, Vx, Vas, Var, Vg, b, scale, bias,
                 Wsn, Wrn):
    return pl.pallas_call(
        _node_body,
        grid=(GN,),
        in_specs=[_blk(BN), _blk(BN), _blk(BN), _blk(BN), _blk(BN),
                  _full((1, D)),
                  _full((D, D)), _full((D, D)), _full((D, D)), _full((D, D)),
                  _full((1, D)), _full((1, D)), _full((1, D)),
                  _full((D, D)), _full((D, D))],
        out_specs=[_blk(BN), _full((1, D)), _blk(BN), _blk(BN)],
        out_shape=[jax.ShapeDtypeStruct((NN, D), jnp.float32),
                   jax.ShapeDtypeStruct((1, D), jnp.float32),
                   jax.ShapeDtypeStruct((NN, D), jnp.float32),
                   jax.ShapeDtypeStruct((NN, D), jnp.float32)],
        compiler_params=pltpu.CompilerParams(dimension_semantics=("arbitrary",)),
    )(x, asa, asb, ara, arb, g, Vx, Vas, Var, Vg, b, scale, bias, Wsn, Wrn)


def _node_last_body(x_ref, asa_ref, asb_ref, ara_ref, arb_ref, g_ref,
                    Vx, Vas, Var, Vg, b, scale, bias, x_out, agg_ref):
    c = _dot(g_ref[...], Vg[...]) + b[...]
    h = (_dot(x_ref[...], Vx[...])
         + _dot(asa_ref[...] + asb_ref[...], Vas[...])
         + _dot(ara_ref[...] + arb_ref[...], Var[...]) + c)
    xn = _ln_relu(h, scale[...], bias[...])
    x_out[...] = xn

    @pl.when(pl.program_id(0) == 0)
    def _():
        agg_ref[...] = jnp.zeros_like(agg_ref)

    agg_ref[...] += jnp.sum(xn, axis=0, keepdims=True)


def _node_update_last(x, asa, asb, ara, arb, g, Vx, Vas, Var, Vg, b, scale,
                      bias):
    return pl.pallas_call(
        _node_last_body,
        grid=(GN,),
        in_specs=[_blk(BN), _blk(BN), _blk(BN), _blk(BN), _blk(BN),
                  _full((1, D)),
                  _full((D, D)), _full((D, D)), _full((D, D)), _full((D, D)),
                  _full((1, D)), _full((1, D)), _full((1, D))],
        out_specs=[_blk(BN), _full((1, D))],
        out_shape=[jax.ShapeDtypeStruct((NN, D), jnp.float32),
                   jax.ShapeDtypeStruct((1, D), jnp.float32)],
        compiler_params=pltpu.CompilerParams(dimension_semantics=("arbitrary",)),
    )(x, asa, asb, ara, arb, g, Vx, Vas, Var, Vg, b, scale, bias)


def _global_body(na_ref, eaa_ref, eab_ref, g_ref, W_ref, b, scale, bias,
                 out_ref):
    W = W_ref[...]
    h = (_dot(na_ref[...], W[0:D]) + _dot(eaa_ref[...] + eab_ref[...],
                                          W[D:2 * D])
         + _dot(g_ref[...], W[2 * D:3 * D]) + b[...])
    out_ref[...] = _ln_relu(h, scale[...], bias[...])


def _global_update(na, eaa, eab, g, W, b, scale, bias):
    return pl.pallas_call(
        _global_body,
        out_shape=jax.ShapeDtypeStruct((1, D), jnp.float32),
    )(na, eaa, eab, g, W, b, scale, bias)


def _decode_body(g_ref, Wd, bd, out_ref):
    out_ref[...] = _dot(g_ref[...], Wd[...]) + bd[...]


def _decode(g, Wd, bd):
    return pl.pallas_call(
        _decode_body,
        out_shape=jax.ShapeDtypeStruct((1, D), jnp.float32),
    )(g, Wd, bd)


def kernel(nodes, edges, senders, receivers, node_embed, edge_embed,
           step_params, decoder, train=False):
    cpta = EA // (NS * CHUNK)
    cptb = EB // (NS * CHUNK)
    sA = senders[:EA].reshape(NS, cpta, CHUNK)
    sB = senders[EA:].reshape(NS, cptb, CHUNK)
    rA = receivers[:EA].reshape(NS, cpta, CHUNK)
    rB = receivers[EA:].reshape(NS, cptb, CHUNK)
    zeros = jnp.zeros((NN, D), jnp.float32)
    g = jnp.zeros((1, D), jnp.float32)

    def row(v):
        return v.reshape(1, D)

    W0 = step_params[0]['edge'][0]
    x, xs, xr = _embed_nodes(nodes, node_embed[0], row(node_embed[1]),
                             W0[D:2 * D], W0[2 * D:3 * D])
    eA = _embed_edges(edges[:EA], edge_embed[0], row(edge_embed[1]))
    eB = _embed_edges(edges[EA:], edge_embed[0], row(edge_embed[1]))

    for t in range(len(step_params)):
        ep = step_params[t]['edge']
        npar = step_params[t]['node']
        gp = step_params[t]['global']
        W = ep[0]
        eb_args = (g, W[0:D], W[3 * D:4 * D],
                   row(ep[1]), row(ep[2]), row(ep[3]))
        gsA, grA = _sc_gather_a(xs, xr, sA, rA)
        gsB, grB = _sc_gather_b(xs, xr, sB, rB)
        eA, eaA = _edge_update(eA, gsA, grA, *eb_args)
        asA, arA = _sc_scatter_a(eA, sA, rA, zeros)
        eB, eaB = _edge_update(eB, gsB, grB, *eb_args)
        asB, arB = _sc_scatter_b(eB, sB, rB, zeros)
        V = npar[0]
        if t + 1 < len(step_params):
            Wn = step_params[t + 1]['edge'][0]
            x, n_agg, xs, xr = _node_update(
                x, asA, asB, arA, arB, g, V[0:D], V[D:2 * D], V[2 * D:3 * D],
                V[3 * D:4 * D], row(npar[1]), row(npar[2]), row(npar[3]),
                Wn[D:2 * D], Wn[2 * D:3 * D])
        else:
            x, n_agg = _node_update_last(
                x, asA, asB, arA, arB, g, V[0:D], V[D:2 * D], V[2 * D:3 * D],
                V[3 * D:4 * D], row(npar[1]), row(npar[2]), row(npar[3]))
        g = _global_update(n_agg, eaA, eaB, g, gp[0], row(gp[1]), row(gp[2]),
                           row(gp[3]))

    return _decode(g, decoder[0], row(decoder[1]))


# speedup vs baseline: 4.4612x; 1.0263x over previous
import jax
import jax.numpy as jnp
from jax import lax
from jax.experimental import pallas as pl
from jax.experimental.pallas import tpu as pltpu
from jax.experimental.pallas import tpu_sc as plsc

NN = 10000
NE = 160000
D = 128
EPS = 1e-6

NC, NS = 2, 16
CHUNK = 80
EA = 96000
EB = NE - EA
K = 5
KC = K * CHUNK
NPT8 = (NN // NS) // 8 * 8
NREM = NN - NS * NPT8

_SC_MESH = plsc.VectorSubcoreMesh(core_axis_name="c", subcore_axis_name="s",
                                  num_cores=NC, num_subcores=NS)


def _pipeline(n_groups, fire_in, drain_in, fire_out, drain_out):
    fire_in(0, 0)

    def body(p, carry):
        drain_in(0)
        fire_out(2 * p, 0)

        @pl.when(p > 0)
        def _():
            drain_out(1)

        fire_in(2 * p + 1, 1)
        drain_in(1)
        fire_out(2 * p + 1, 1)
        drain_out(0)
        fire_in(2 * p + 2, 0)
        return carry

    lax.fori_loop(0, (n_groups - 1) // 2, body, 0)
    if n_groups % 2:
        drain_in(0)
        fire_out(n_groups - 1, 0)
        drain_out(1)
        drain_out(0)
    else:
        drain_in(0)
        fire_out(n_groups - 2, 0)
        drain_out(1)
        fire_in(n_groups - 1, 1)
        drain_in(1)
        fire_out(n_groups - 1, 1)
        drain_out(0)
        drain_out(1)


def _make_sc_gather(n_edges):
    cpt = n_edges // (NS * CHUNK)
    ng = cpt // K

    def body(xs_hbm, xr_hbm, s_hbm, r_hbm, gs_hbm, gr_hbm,
             idx_v, rows_a, rows_b, sin_a, sin_b, sout_a, sout_b):
        cid = lax.axis_index("c")
        sid = lax.axis_index("s")
        rows = (rows_a, rows_b)
        sin = (sin_a, sin_b)
        sout = (sout_a, sout_b)

        def run(tab, idx3d, out):
            pltpu.sync_copy(idx3d.at[sid], idx_v)

            def fire_in(g, i):
                for k in range(K):
                    pltpu.async_copy(tab.at[idx_v.at[g * K + k]],
                                     rows[i].at[pl.ds(k * CHUNK, CHUNK)],
                                     sin[i])

            def drain_in(i):
                pltpu.make_async_copy(tab.at[pl.ds(0, KC)], rows[i],
                                      sin[i]).wait()

            def fire_out(g, i):
                pltpu.async_copy(rows[i],
                                 out.at[pl.ds((sid * ng + g) * KC, KC)],
                                 sout[i])

            def drain_out(i):
                pltpu.make_async_copy(rows[i], out.at[pl.ds(0, KC)],
                                      sout[i]).wait()

            _pipeline(ng, fire_in, drain_in, fire_out, drain_out)

        @pl.when(cid == 0)
        def _():
            run(xs_hbm, s_hbm, gs_hbm)

        @pl.when(cid == 1)
        def _():
            run(xr_hbm, r_hbm, gr_hbm)

    return pl.kernel(
        body,
        out_type=[jax.ShapeDtypeStruct((n_edges, D), jnp.float32),
                  jax.ShapeDtypeStruct((n_edges, D), jnp.float32)],
        mesh=_SC_MESH,
        scratch_types=[pltpu.VMEM((cpt, CHUNK), jnp.int32),
                       pltpu.VMEM((KC, D), jnp.float32),
                       pltpu.VMEM((KC, D), jnp.float32),
                       pltpu.SemaphoreType.DMA,
                       pltpu.SemaphoreType.DMA,
                       pltpu.SemaphoreType.DMA,
                       pltpu.SemaphoreType.DMA],
    )


def _make_sc_scatter(n_edges):
    cpt = n_edges // (NS * CHUNK)

    def body(e_hbm, s_hbm, r_hbm, inis_hbm, inir_hbm, aggs_hbm, aggr_hbm,
             idx_v, rows_a, rows_b, sin_a, sin_b, sout_a, sout_b, acc):
        cid = lax.axis_index("c")
        sid = lax.axis_index("s")
        rows = (rows_a, rows_b)
        sin = (sin_a, sin_b)
        sout = (sout_a, sout_b)

        def init(ini_hbm):
            pltpu.sync_copy(ini_hbm.at[pl.ds(sid * NPT8, NPT8)],
                            acc.at[pl.ds(sid * NPT8, NPT8)])

            @pl.when(sid == NS - 1)
            def _():
                pltpu.sync_copy(ini_hbm.at[pl.ds(NS * NPT8, NREM)],
                                acc.at[pl.ds(NS * NPT8, NREM)])

        @pl.when(cid == 0)
        def _():
            init(inis_hbm)

        @pl.when(cid == 1)
        def _():
            init(inir_hbm)

        plsc.subcore_barrier()

        def run(idx3d, out):
            pltpu.sync_copy(idx3d.at[sid], idx_v)

            def fire_in(j, i):
                pltpu.async_copy(
                    e_hbm.at[pl.ds((sid * cpt + j) * CHUNK, CHUNK)],
                    rows[i], sin[i])

            def drain_in(i):
                pltpu.make_async_copy(e_hbm.at[pl.ds(0, CHUNK)], rows[i],
                                      sin[i]).wait()

            def fire_out(j, i):
                pltpu.async_copy(rows[i], acc.at[idx_v.at[j]], sout[i],
                                 add=True)

            def drain_out(i):
                pltpu.make_async_copy(e_hbm.at[pl.ds(0, CHUNK)], rows[i],
                                      sout[i]).wait()

            _pipeline(cpt, fire_in, drain_in, fire_out, drain_out)
            plsc.subcore_barrier()
            pltpu.sync_copy(acc.at[pl.ds(sid * NPT8, NPT8)],
                            out.at[pl.ds(sid * NPT8, NPT8)])

            @pl.when(sid == NS - 1)
            def _():
                pltpu.sync_copy(acc.at[pl.ds(NS * NPT8, NREM)],
                                out.at[pl.ds(NS * NPT8, NREM)])

        @pl.when(cid == 0)
        def _():
            run(s_hbm, aggs_hbm)

        @pl.when(cid == 1)
        def _():
            run(r_hbm, aggr_hbm)

    return pl.kernel(
        body,
        out_type=[jax.ShapeDtypeStruct((NN, D), jnp.float32),
                  jax.ShapeDtypeStruct((NN, D), jnp.float32)],
        mesh=_SC_MESH,
        scratch_types=[pltpu.VMEM((cpt, CHUNK), jnp.int32),
                       pltpu.VMEM((CHUNK, D), jnp.float32),
                       pltpu.VMEM((CHUNK, D), jnp.float32),
                       pltpu.SemaphoreType.DMA,
                       pltpu.SemaphoreType.DMA,
                       pltpu.SemaphoreType.DMA,
                       pltpu.SemaphoreType.DMA,
                       pltpu.VMEM_SHARED((NN, D), jnp.float32)],
    )


_sc_gather_a = _make_sc_gather(EA)
_sc_gather_b = _make_sc_gather(EB)
_sc_scatter_a = _make_sc_scatter(EA)
_sc_scatter_b = _make_sc_scatter(EB)


def _ln_relu(h, scale, bias):
    mean = jnp.mean(h, axis=-1, keepdims=True)
    var = jnp.mean(jnp.square(h - mean), axis=-1, keepdims=True)
    return jnp.maximum((h - mean) * lax.rsqrt(var + EPS) * scale + bias, 0.0)


def _dot(a, b):
    return jnp.dot(a, b, preferred_element_type=jnp.float32)


BE = 2000
BN = 2000
GN = NN // BN

_blk = lambda r: pl.BlockSpec((r, D), lambda i: (i, 0))
_full = lambda s: pl.BlockSpec(s, lambda i: (0, 0))


def _embed_nodes_body(n_ref, Wn, bn, Ws, Wr, x_out, xs_out, xr_out):
    x = _dot(n_ref[...], Wn[...]) + bn[...]
    x_out[...] = x
    xs_out[...] = _dot(x, Ws[...])
    xr_out[...] = _dot(x, Wr[...])


def _embed_nodes(nodes, Wn, bn, Ws, Wr):
    return pl.pallas_call(
        _embed_nodes_body,
        grid=(GN,),
        in_specs=[_blk(BN), _full((D, D)), _full((1, D)), _full((D, D)), _full((D, D))],
        out_specs=[_blk(BN), _blk(BN), _blk(BN)],
        out_shape=[jax.ShapeDtypeStruct((NN, D), jnp.float32)] * 3,
    )(nodes, Wn, bn, Ws, Wr)


def _embed_edges_body(e_ref, We, be, out_ref):
    out_ref[...] = _dot(e_ref[...], We[...]) + be[...]


def _embed_edges(edges, We, be):
    rows, de = edges.shape
    return pl.pallas_call(
        _embed_edges_body,
        grid=(rows // BE,),
        in_specs=[pl.BlockSpec((BE, de), lambda i: (i, 0)),
                  _full((de, D)), _full((1, D))],
        out_specs=_blk(BE),
        out_shape=jax.ShapeDtypeStruct((rows, D), jnp.float32),
    )(edges, We, be)


def _edge_body(e_ref, gs_ref, gr_ref, g_ref, We, Wg, b, scale, bias,
               out_ref, agg_ref):
    c = _dot(g_ref[...], Wg[...]) + b[...]
    h = _dot(e_ref[...], We[...]) + gs_ref[...] + gr_ref[...] + c
    en = _ln_relu(h, scale[...], bias[...])
    out_ref[...] = en

    @pl.when(pl.program_id(0) == 0)
    def _():
        agg_ref[...] = jnp.zeros_like(agg_ref)

    agg_ref[...] += jnp.sum(en, axis=0, keepdims=True)


def _edge0_body(eraw_ref, gs_ref, gr_ref, g_ref, Wemb, bemb, We, Wg, b,
                scale, bias, out_ref, agg_ref):
    m = _dot(Wemb[...], We[...])
    c = _dot(bemb[...], We[...]) + _dot(g_ref[...], Wg[...]) + b[...]
    h = _dot(eraw_ref[...], m) + gs_ref[...] + gr_ref[...] + c
    en = _ln_relu(h, scale[...], bias[...])
    out_ref[...] = en

    @pl.when(pl.program_id(0) == 0)
    def _():
        agg_ref[...] = jnp.zeros_like(agg_ref)

    agg_ref[...] += jnp.sum(en, axis=0, keepdims=True)


def _edge_update0(eraw, gs, gr, g, Wemb, bemb, We, Wg, b, scale, bias):
    rows, de = eraw.shape
    return pl.pallas_call(
        _edge0_body,
        grid=(rows // BE,),
        in_specs=[pl.BlockSpec((BE, de), lambda i: (i, 0)),
                  _blk(BE), _blk(BE), _full((1, D)),
                  _full((de, D)), _full((1, D)),
                  _full((D, D)), _full((D, D)),
                  _full((1, D)), _full((1, D)), _full((1, D))],
        out_specs=[_blk(BE), _full((1, D))],
        out_shape=[jax.ShapeDtypeStruct((rows, D), jnp.float32),
                   jax.ShapeDtypeStruct((1, D), jnp.float32)],
        compiler_params=pltpu.CompilerParams(dimension_semantics=("arbitrary",)),
    )(eraw, gs, gr, g, Wemb, bemb, We, Wg, b, scale, bias)


def _edge_update(e, gs, gr, g, We, Wg, b, scale, bias):
    rows = e.shape[0]
    return pl.pallas_call(
        _edge_body,
        grid=(rows // BE,),
        in_specs=[_blk(BE), _blk(BE), _blk(BE), _full((1, D)),
                  _full((D, D)), _full((D, D)),
                  _full((1, D)), _full((1, D)), _full((1, D))],
        out_specs=[_blk(BE), _full((1, D))],
        out_shape=[jax.ShapeDtypeStruct((rows, D), jnp.float32),
                   jax.ShapeDtypeStruct((1, D), jnp.float32)],
        compiler_params=pltpu.CompilerParams(dimension_semantics=("arbitrary",)),
    )(e, gs, gr, g, We, Wg, b, scale, bias)


def _node_body(x_ref, as_ref, ar_ref, g_ref, Vx, Vas, Var, Vg, b, scale, bias,
               Wsn, Wrn, x_out, agg_ref, xs_out, xr_out):
    c = _dot(g_ref[...], Vg[...]) + b[...]
    h = (_dot(x_ref[...], Vx[...]) + _dot(as_ref[...], Vas[...])
         + _dot(ar_ref[...], Var[...]) + c)
    xn = _ln_relu(h, scale[...], bias[...])
    x_out[...] = xn

    @pl.when(pl.program_id(0) == 0)
    def _():
        agg_ref[...] = jnp.zeros_like(agg_ref)

    agg_ref[...] += jnp.sum(xn, axis=0, keepdims=True)
    xs_out[...] = _dot(xn, Wsn[...])
    xr_out[...] = _dot(xn, Wrn[...])


def _node_update(x, aggs, aggr, g, Vx, Vas, Var, Vg, b, scale, bias, Wsn, Wrn):
    return pl.pallas_call(
        _node_body,
        grid=(GN,),
        in_specs=[_blk(BN), _blk(BN), _blk(BN), _full((1, D)),
                  _full((D, D)), _full((D, D)), _full((D, D)), _full((D, D)),
                  _full((1, D)), _full((1, D)), _full((1, D)),
                  _full((D, D)), _full((D, D))],
        out_specs=[_blk(BN), _full((1, D)), _blk(BN), _blk(BN)],
        out_shape=[jax.ShapeDtypeStruct((NN, D), jnp.float32),
                   jax.ShapeDtypeStruct((1, D), jnp.float32),
                   jax.ShapeDtypeStruct((NN, D), jnp.float32),
                   jax.ShapeDtypeStruct((NN, D), jnp.float32)],
        compiler_params=pltpu.CompilerParams(dimension_semantics=("arbitrary",)),
    )(x, aggs, aggr, g, Vx, Vas, Var, Vg, b, scale, bias, Wsn, Wrn)


def _node_last_body(x_ref, as_ref, ar_ref, g_ref, Vx, Vas, Var, Vg, b, scale,
                    bias, agg_ref):
    c = _dot(g_ref[...], Vg[...]) + b[...]
    h = (_dot(x_ref[...], Vx[...]) + _dot(as_ref[...], Vas[...])
         + _dot(ar_ref[...], Var[...]) + c)
    xn = _ln_relu(h, scale[...], bias[...])

    @pl.when(pl.program_id(0) == 0)
    def _():
        agg_ref[...] = jnp.zeros_like(agg_ref)

    agg_ref[...] += jnp.sum(xn, axis=0, keepdims=True)


def _node_update_last(x, aggs, aggr, g, Vx, Vas, Var, Vg, b, scale, bias):
    return pl.pallas_call(
        _node_last_body,
        grid=(GN,),
        in_specs=[_blk(BN), _blk(BN), _blk(BN), _full((1, D)),
                  _full((D, D)), _full((D, D)), _full((D, D)), _full((D, D)),
                  _full((1, D)), _full((1, D)), _full((1, D))],
        out_specs=_full((1, D)),
        out_shape=jax.ShapeDtypeStruct((1, D), jnp.float32),
        compiler_params=pltpu.CompilerParams(dimension_semantics=("arbitrary",)),
    )(x, aggs, aggr, g, Vx, Vas, Var, Vg, b, scale, bias)


def _global_body(na_ref, eaa_ref, eab_ref, g_ref, W_ref, b, scale, bias,
                 out_ref):
    W = W_ref[...]
    h = (_dot(na_ref[...], W[0:D]) + _dot(eaa_ref[...] + eab_ref[...],
                                          W[D:2 * D])
         + _dot(g_ref[...], W[2 * D:3 * D]) + b[...])
    out_ref[...] = _ln_relu(h, scale[...], bias[...])


def _global_update(na, eaa, eab, g, W, b, scale, bias):
    return pl.pallas_call(
        _global_body,
        out_shape=jax.ShapeDtypeStruct((1, D), jnp.float32),
    )(na, eaa, eab, g, W, b, scale, bias)


def _decode_body(g_ref, Wd, bd, out_ref):
    out_ref[...] = _dot(g_ref[...], Wd[...]) + bd[...]


def _decode(g, Wd, bd):
    return pl.pallas_call(
        _decode_body,
        out_shape=jax.ShapeDtypeStruct((1, D), jnp.float32),
    )(g, Wd, bd)


def kernel(nodes, edges, senders, receivers, node_embed, edge_embed,
           step_params, decoder, train=False):
    cpta = EA // (NS * CHUNK)
    cptb = EB // (NS * CHUNK)
    sA = senders[:EA].reshape(NS, cpta, CHUNK)
    sB = senders[EA:].reshape(NS, cptb, CHUNK)
    rA = receivers[:EA].reshape(NS, cpta, CHUNK)
    rB = receivers[EA:].reshape(NS, cptb, CHUNK)
    zeros = jnp.zeros((NN, D), jnp.float32)
    g = jnp.zeros((1, D), jnp.float32)

    def row(v):
        return v.reshape(1, D)

    W0 = step_params[0]['edge'][0]
    x, xs, xr = _embed_nodes(nodes, node_embed[0], row(node_embed[1]),
                             W0[D:2 * D], W0[2 * D:3 * D])
    eA = edges[:EA]
    eB = edges[EA:]

    for t in range(len(step_params)):
        ep = step_params[t]['edge']
        npar = step_params[t]['node']
        gp = step_params[t]['global']
        W = ep[0]
        eb_args = (g, W[0:D], W[3 * D:4 * D],
                   row(ep[1]), row(ep[2]), row(ep[3]))
        gsA, grA = _sc_gather_a(xs, xr, sA, rA)
        gsB, grB = _sc_gather_b(xs, xr, sB, rB)
        if t == 0:
            e0_args = (g, edge_embed[0], row(edge_embed[1]), W[0:D],
                       W[3 * D:4 * D], row(ep[1]), row(ep[2]), row(ep[3]))
            eA, eaA = _edge_update0(eA, gsA, grA, *e0_args)
            asA, arA = _sc_scatter_a(eA, sA, rA, zeros, zeros)
            eB, eaB = _edge_update0(eB, gsB, grB, *e0_args)
        else:
            eA, eaA = _edge_update(eA, gsA, grA, *eb_args)
            asA, arA = _sc_scatter_a(eA, sA, rA, zeros, zeros)
            eB, eaB = _edge_update(eB, gsB, grB, *eb_args)
        aggs, aggr = _sc_scatter_b(eB, sB, rB, asA, arA)
        V = npar[0]
        if t + 1 < len(step_params):
            Wn = step_params[t + 1]['edge'][0]
            x, n_agg, xs, xr = _node_update(
                x, aggs, aggr, g, V[0:D], V[D:2 * D], V[2 * D:3 * D],
                V[3 * D:4 * D], row(npar[1]), row(npar[2]), row(npar[3]),
                Wn[D:2 * D], Wn[2 * D:3 * D])
        else:
            n_agg = _node_update_last(
                x, aggs, aggr, g, V[0:D], V[D:2 * D], V[2 * D:3 * D],
                V[3 * D:4 * D], row(npar[1]), row(npar[2]), row(npar[3]))
        g = _global_update(n_agg, eaA, eaB, g, gp[0], row(gp[1]), row(gp[2]),
                           row(gp[3]))

    return _decode(g, decoder[0], row(decoder[1]))
